# baseline (speedup 1.0000x reference)
import functools

import jax
import jax.numpy as jnp
from jax import lax
from jax.experimental import pallas as pl
from jax.experimental.pallas import tpu as pltpu
from jax.experimental.pallas import tpu_sc as plsc

D = 32
B = 16384
SEQ = 20
TEXT_V = 10000
NC, NS = 2, 16
NW = NC * NS
BPW = B // NW
CH = 4
TPC = CH * SEQ
NCHUNK = BPW // CH
NBUF = 2
L = 16
TCHUNK = 128


def _body(title_idx_hbm, tok_hbm, title_tab_hbm, text_tab_hbm, out_hbm,
          title_idx_v, tok_idx_v, mask_v, title_rows_v, tok_rows_v, out_v,
          sem_a, sem_b, sem_title):
    wid = lax.axis_index("s") * NC + lax.axis_index("c")
    base = wid * BPW
    sems = [sem_a, sem_b]

    pltpu.sync_copy(title_idx_hbm.at[pl.ds(base, BPW)], title_idx_v)

    for j in range(BPW // TCHUNK):
        pltpu.async_copy(
            title_tab_hbm.at[title_idx_v.at[pl.ds(j * TCHUNK, TCHUNK)]],
            title_rows_v.at[pl.ds(j * TCHUNK, TCHUNK)], sem_title)

    pltpu.sync_copy(tok_hbm.at[pl.ds(base * SEQ, BPW * SEQ)], tok_idx_v)

    @pl.loop(0, BPW * SEQ // L, unroll=4)
    def prepass(i):
        tv = tok_idx_v[pl.ds(i * L, L)]
        mn = jnp.minimum(tv, 1)
        tok_idx_v[pl.ds(i * L, L)] = tv + (1 - mn) * TEXT_V
        mask_v[pl.ds(i * L, L)] = mn.astype(jnp.float32)

    def issue(b, c):
        pltpu.async_copy(
            text_tab_hbm.at[tok_idx_v.at[pl.ds(c * TPC, TPC)]],
            tok_rows_v.at[b], sems[b])

    def drain(b):
        pltpu.make_async_copy(
            text_tab_hbm.at[pl.ds(0, TPC)], tok_rows_v.at[b], sems[b]).wait()

    for b in range(NBUF):
        issue(b, b)

    for j in range(BPW // TCHUNK):
        pltpu.make_async_copy(
            title_tab_hbm.at[pl.ds(0, TCHUNK)],
            title_rows_v.at[pl.ds(j * TCHUNK, TCHUNK)], sem_title).wait()

    @pl.loop(0, NCHUNK, step=NBUF)
    def outer(c0):
        for b in range(NBUF):
            c = c0 + b
            drain(b)
            ms = []
            for k in range(TPC // L):
                ms.append(mask_v[pl.ds(c * TPC + k * L, L)])
            for r in range(CH):
                acc0 = jnp.zeros((L,), jnp.float32)
                acc1 = jnp.zeros((L,), jnp.float32)
                cnt = jnp.float32(0.0)
                for t in range(SEQ):
                    f = r * SEQ + t
                    acc0 = acc0 + tok_rows_v[b, f, 0:L]
                    acc1 = acc1 + tok_rows_v[b, f, L:2 * L]
                    cnt = cnt + ms[f // L][f % L]
                den_v = jnp.maximum(jnp.full((L,), cnt, jnp.float32),
                                    jnp.float32(1e-9))
                inv_v = jnp.float32(1.0) / den_v
                row = c * CH + r
                out_v[row, 2 * L:3 * L] = acc0 * inv_v
                out_v[row, 3 * L:4 * L] = acc1 * inv_v
                out_v[row, 0:L] = title_rows_v[row, 0:L]
                out_v[row, L:2 * L] = title_rows_v[row, L:2 * L]
            nxt = c + NBUF
            @pl.when(nxt < NCHUNK)
            def _():
                issue(b, nxt)

    pltpu.sync_copy(out_v, out_hbm.at[pl.ds(base, BPW)])


_sc_call = pl.kernel(
    _body,
    out_type=jax.ShapeDtypeStruct((B, 2 * D), jnp.float32),
    mesh=plsc.VectorSubcoreMesh(
        core_axis_name="c", subcore_axis_name="s",
        num_cores=NC, num_subcores=NS),
    scratch_types=[
        pltpu.VMEM((BPW,), jnp.int32),
        pltpu.VMEM((BPW * SEQ,), jnp.int32),
        pltpu.VMEM((BPW * SEQ,), jnp.float32),
        pltpu.VMEM((BPW, D), jnp.float32),
        pltpu.VMEM((NBUF, TPC, D), jnp.float32),
        pltpu.VMEM((BPW, 2 * D), jnp.float32),
        pltpu.SemaphoreType.DMA,
        pltpu.SemaphoreType.DMA,
        pltpu.SemaphoreType.DMA,
    ],
    compiler_params=pltpu.CompilerParams(use_tc_tiling_on_sc=False),
)


@jax.jit
def kernel(title_idx, text_tokens, title_table, text_table):
    ti = title_idx.astype(jnp.int32)
    tok = text_tokens.astype(jnp.int32).reshape(-1)
    text_aug = jnp.concatenate(
        [text_table, jnp.zeros((1, D), jnp.float32)], axis=0)
    return _sc_call(ti, tok, title_table, text_aug)


# speedup vs baseline: 1.1795x; 1.1795x over previous
import functools

import jax
import jax.numpy as jnp
from jax import lax
from jax.experimental import pallas as pl
from jax.experimental.pallas import tpu as pltpu
from jax.experimental.pallas import tpu_sc as plsc

D = 32
B = 16384
SEQ = 20
TEXT_V = 10000
NC, NS = 2, 16
NW = NC * NS
BPW = B // NW
CH = 4
TPC = CH * SEQ
NCHUNK = BPW // CH
NBUF = 4
L = 16
TCHUNK = 128


def _body(title_idx_hbm, tok_hbm, title_tab_hbm, text_tab_hbm, out_hbm,
          title_idx_v, tok_idx_v, mask_v, title_rows_v, tok_rows_v, out_v,
          sem_a, sem_b, sem_c, sem_d, sem_title):
    wid = lax.axis_index("s") * NC + lax.axis_index("c")
    base = wid * BPW
    sems = [sem_a, sem_b, sem_c, sem_d]

    pltpu.sync_copy(title_idx_hbm.at[pl.ds(base, BPW)], title_idx_v)

    for j in range(BPW // TCHUNK):
        pltpu.async_copy(
            title_tab_hbm.at[title_idx_v.at[pl.ds(j * TCHUNK, TCHUNK)]],
            title_rows_v.at[pl.ds(j * TCHUNK, TCHUNK)], sem_title)

    pltpu.sync_copy(tok_hbm.at[pl.ds(base * SEQ, BPW * SEQ)], tok_idx_v)

    @pl.loop(0, BPW * SEQ // L, unroll=4)
    def prepass(i):
        tv = tok_idx_v[pl.ds(i * L, L)]
        mn = jnp.minimum(tv, 1)
        tok_idx_v[pl.ds(i * L, L)] = tv + (1 - mn) * TEXT_V
        mask_v[pl.ds(i * L, L)] = mn.astype(jnp.float32)

    def issue(b, c):
        pltpu.async_copy(
            text_tab_hbm.at[tok_idx_v.at[pl.ds(c * TPC, TPC)]],
            tok_rows_v.at[b], sems[b])

    def drain(b):
        pltpu.make_async_copy(
            text_tab_hbm.at[pl.ds(0, TPC)], tok_rows_v.at[b], sems[b]).wait()

    for b in range(NBUF):
        issue(b, b)

    for j in range(BPW // TCHUNK):
        pltpu.make_async_copy(
            title_tab_hbm.at[pl.ds(0, TCHUNK)],
            title_rows_v.at[pl.ds(j * TCHUNK, TCHUNK)], sem_title).wait()

    @pl.loop(0, NCHUNK, step=NBUF)
    def outer(c0):
        for b in range(NBUF):
            c = c0 + b
            drain(b)
            ms = []
            for k in range(TPC // L):
                ms.append(mask_v[pl.ds(c * TPC + k * L, L)])
            for r in range(CH):
                acc0 = jnp.zeros((L,), jnp.float32)
                acc1 = jnp.zeros((L,), jnp.float32)
                cnt = jnp.float32(0.0)
                for t in range(SEQ):
                    f = r * SEQ + t
                    acc0 = acc0 + tok_rows_v[b, f, 0:L]
                    acc1 = acc1 + tok_rows_v[b, f, L:2 * L]
                    cnt = cnt + ms[f // L][f % L]
                den_v = jnp.maximum(jnp.full((L,), cnt, jnp.float32),
                                    jnp.float32(1e-9))
                inv_v = jnp.float32(1.0) / den_v
                row = c * CH + r
                out_v[row, 2 * L:3 * L] = acc0 * inv_v
                out_v[row, 3 * L:4 * L] = acc1 * inv_v
                out_v[row, 0:L] = title_rows_v[row, 0:L]
                out_v[row, L:2 * L] = title_rows_v[row, L:2 * L]
            nxt = c + NBUF
            @pl.when(nxt < NCHUNK)
            def _():
                issue(b, nxt)

    pltpu.sync_copy(out_v, out_hbm.at[pl.ds(base, BPW)])


_sc_call = pl.kernel(
    _body,
    out_type=jax.ShapeDtypeStruct((B, 2 * D), jnp.float32),
    mesh=plsc.VectorSubcoreMesh(
        core_axis_name="c", subcore_axis_name="s",
        num_cores=NC, num_subcores=NS),
    scratch_types=[
        pltpu.VMEM((BPW,), jnp.int32),
        pltpu.VMEM((BPW * SEQ,), jnp.int32),
        pltpu.VMEM((BPW * SEQ,), jnp.float32),
        pltpu.VMEM((BPW, D), jnp.float32),
        pltpu.VMEM((NBUF, TPC, D), jnp.float32),
        pltpu.VMEM((BPW, 2 * D), jnp.float32),
        pltpu.SemaphoreType.DMA,
        pltpu.SemaphoreType.DMA,
        pltpu.SemaphoreType.DMA,
        pltpu.SemaphoreType.DMA,
        pltpu.SemaphoreType.DMA,
    ],
    compiler_params=pltpu.CompilerParams(use_tc_tiling_on_sc=False),
)


@jax.jit
def kernel(title_idx, text_tokens, title_table, text_table):
    ti = title_idx.astype(jnp.int32)
    tok = text_tokens.astype(jnp.int32).reshape(-1)
    text_aug = jnp.concatenate(
        [text_table, jnp.zeros((1, D), jnp.float32)], axis=0)
    return _sc_call(ti, tok, title_table, text_aug)


# speedup vs baseline: 1.2539x; 1.0631x over previous
import functools

import jax
import jax.numpy as jnp
from jax import lax
from jax.experimental import pallas as pl
from jax.experimental.pallas import tpu as pltpu
from jax.experimental.pallas import tpu_sc as plsc

D = 32
B = 16384
SEQ = 20
TEXT_V = 10000
NC, NS = 2, 16
NW = NC * NS
BPW = B // NW
CH = 4
TPC = CH * SEQ
NCHUNK = BPW // CH
NBUF = 4
L = 16
TCHUNK = 128


def _body(title_idx_hbm, tok_hbm, title_tab_hbm, text_tab_hbm, out_hbm,
          title_idx_v, tok_idx_v, mask_v, title_rows_v, tok_rows_v, out_v,
          text_tab_sp, sem_a, sem_b, sem_c, sem_d, sem_title):
    wid = lax.axis_index("s") * NC + lax.axis_index("c")
    base = wid * BPW
    sems = [sem_a, sem_b, sem_c, sem_d]

    @pl.when(lax.axis_index("s") == 0)
    def _stage_table():
        pltpu.sync_copy(text_tab_hbm, text_tab_sp)

    pltpu.sync_copy(title_idx_hbm.at[pl.ds(base, BPW)], title_idx_v)

    for j in range(BPW // TCHUNK):
        pltpu.async_copy(
            title_tab_hbm.at[title_idx_v.at[pl.ds(j * TCHUNK, TCHUNK)]],
            title_rows_v.at[pl.ds(j * TCHUNK, TCHUNK)], sem_title)

    pltpu.sync_copy(tok_hbm.at[pl.ds(base * SEQ, BPW * SEQ)], tok_idx_v)

    @pl.loop(0, BPW * SEQ // L, unroll=4)
    def prepass(i):
        tv = tok_idx_v[pl.ds(i * L, L)]
        mn = jnp.minimum(tv, 1)
        tok_idx_v[pl.ds(i * L, L)] = tv + (1 - mn) * TEXT_V
        mask_v[pl.ds(i * L, L)] = mn.astype(jnp.float32)

    def issue(b, c):
        pltpu.async_copy(
            text_tab_sp.at[tok_idx_v.at[pl.ds(c * TPC, TPC)]],
            tok_rows_v.at[b], sems[b])

    def drain(b):
        pltpu.make_async_copy(
            text_tab_hbm.at[pl.ds(0, TPC)], tok_rows_v.at[b], sems[b]).wait()

    plsc.subcore_barrier()

    for b in range(NBUF):
        issue(b, b)

    for j in range(BPW // TCHUNK):
        pltpu.make_async_copy(
            title_tab_hbm.at[pl.ds(0, TCHUNK)],
            title_rows_v.at[pl.ds(j * TCHUNK, TCHUNK)], sem_title).wait()

    @pl.loop(0, NCHUNK, step=NBUF)
    def outer(c0):
        for b in range(NBUF):
            c = c0 + b
            drain(b)
            ms = []
            for k in range(TPC // L):
                ms.append(mask_v[pl.ds(c * TPC + k * L, L)])
            for r in range(CH):
                acc0 = jnp.zeros((L,), jnp.float32)
                acc1 = jnp.zeros((L,), jnp.float32)
                cnt = jnp.float32(0.0)
                for t in range(SEQ):
                    f = r * SEQ + t
                    acc0 = acc0 + tok_rows_v[b, f, 0:L]
                    acc1 = acc1 + tok_rows_v[b, f, L:2 * L]
                    cnt = cnt + ms[f // L][f % L]
                den_v = jnp.maximum(jnp.full((L,), cnt, jnp.float32),
                                    jnp.float32(1e-9))
                inv_v = jnp.float32(1.0) / den_v
                row = c * CH + r
                out_v[row, 2 * L:3 * L] = acc0 * inv_v
                out_v[row, 3 * L:4 * L] = acc1 * inv_v
                out_v[row, 0:L] = title_rows_v[row, 0:L]
                out_v[row, L:2 * L] = title_rows_v[row, L:2 * L]
            nxt = c + NBUF
            @pl.when(nxt < NCHUNK)
            def _():
                issue(b, nxt)

    pltpu.sync_copy(out_v, out_hbm.at[pl.ds(base, BPW)])


_sc_call = pl.kernel(
    _body,
    out_type=jax.ShapeDtypeStruct((B, 2 * D), jnp.float32),
    mesh=plsc.VectorSubcoreMesh(
        core_axis_name="c", subcore_axis_name="s",
        num_cores=NC, num_subcores=NS),
    scratch_types=[
        pltpu.VMEM((BPW,), jnp.int32),
        pltpu.VMEM((BPW * SEQ,), jnp.int32),
        pltpu.VMEM((BPW * SEQ,), jnp.float32),
        pltpu.VMEM((BPW, D), jnp.float32),
        pltpu.VMEM((NBUF, TPC, D), jnp.float32),
        pltpu.VMEM((BPW, 2 * D), jnp.float32),
        pltpu.VMEM_SHARED((TEXT_V + 1, D), jnp.float32),
        pltpu.SemaphoreType.DMA,
        pltpu.SemaphoreType.DMA,
        pltpu.SemaphoreType.DMA,
        pltpu.SemaphoreType.DMA,
        pltpu.SemaphoreType.DMA,
    ],
    compiler_params=pltpu.CompilerParams(use_tc_tiling_on_sc=False),
)


@jax.jit
def kernel(title_idx, text_tokens, title_table, text_table):
    ti = title_idx.astype(jnp.int32)
    tok = text_tokens.astype(jnp.int32).reshape(-1)
    text_aug = jnp.concatenate(
        [text_table, jnp.zeros((1, D), jnp.float32)], axis=0)
    return _sc_call(ti, tok, title_table, text_aug)


# speedup vs baseline: 1.2901x; 1.0289x over previous
import functools

import jax
import jax.numpy as jnp
from jax import lax
from jax.experimental import pallas as pl
from jax.experimental.pallas import tpu as pltpu
from jax.experimental.pallas import tpu_sc as plsc

D = 32
B = 16384
SEQ = 20
TEXT_V = 10000
V2 = 10008
NP = 8
NC, NS = 2, 16
NW = NC * NS
BPW = B // NW
GR = 2 * BPW
NCHUNK = GR // 16
L = 16
TCHUNK = 128


def _body(title_idx_hbm, tokT_hbm, title_tab_hbm, tab_lo_hbm, tab_hi_hbm,
          out_hbm, tab_v, tokT_v, inv_v, title_idx_v, title_rows_v, txt_blk,
          sem_title, sem_out):
    wid = lax.axis_index("s") * NC + lax.axis_index("c")
    ph = wid % 2
    rbase = (wid // 2) * GR
    tbase = wid * BPW

    @pl.when(ph == 0)
    def _lo():
        pltpu.sync_copy(tab_lo_hbm, tab_v)

    @pl.when(ph == 1)
    def _hi():
        pltpu.sync_copy(tab_hi_hbm, tab_v)

    pltpu.sync_copy(title_idx_hbm.at[pl.ds(tbase, BPW)], title_idx_v)
    for j in range(BPW // TCHUNK):
        pltpu.async_copy(
            title_tab_hbm.at[title_idx_v.at[pl.ds(j * TCHUNK, TCHUNK)]],
            title_rows_v.at[pl.ds(j * TCHUNK, TCHUNK)], sem_title)

    pltpu.sync_copy(tokT_hbm.at[:, pl.ds(rbase, GR)], tokT_v)

    @pl.loop(0, NCHUNK)
    def prepass(rb):
        cnt = jnp.zeros((L,), jnp.float32)
        for t in range(SEQ):
            tv = tokT_v[t, pl.ds(rb * L, L)]
            mn = jnp.minimum(tv, 1)
            tokT_v[t, pl.ds(rb * L, L)] = tv + (1 - mn) * TEXT_V
            cnt = cnt + mn.astype(jnp.float32)
        inv_v[pl.ds(rb * L, L)] = jnp.float32(1.0) / jnp.maximum(
            cnt, jnp.float32(1e-9))

    iota = lax.iota(jnp.int32, L)
    cols = [jnp.full((L,), p, jnp.int32) for p in range(NP)]
    colbase = D + L * ph

    @pl.loop(0, NCHUNK, step=2)
    def pool(rb0):
        for bslot in range(2):
            rb = rb0 + bslot

            @pl.when(rb0 > 0)
            def _drain_slot():
                pltpu.make_async_copy(
                    txt_blk.at[bslot],
                    out_hbm.at[pl.ds(0, L), pl.ds(D, L)], sem_out).wait()

            inv = inv_v[pl.ds(rb * L, L)]
            accs = [jnp.zeros((L,), jnp.float32) for _ in range(2 * NP)]
            for t in range(SEQ):
                tokv = tokT_v[t, pl.ds(rb * L, L)]
                for p in range(NP):
                    g = plsc.load_gather(tab_v, [tokv, cols[p]])
                    lo = plsc.bitcast(g << 16, jnp.float32)
                    hi = plsc.bitcast(g & jnp.int32(-65536), jnp.float32)
                    accs[2 * p] = accs[2 * p] + lo
                    accs[2 * p + 1] = accs[2 * p + 1] + hi
            for d in range(2 * NP):
                plsc.store_scatter(
                    txt_blk.at[bslot],
                    [iota, jnp.full((L,), d, jnp.int32)], accs[d] * inv)
            pltpu.async_copy(
                txt_blk.at[bslot],
                out_hbm.at[pl.ds(rbase + rb * L, L), pl.ds(colbase, L)],
                sem_out)

    for _ in range(2):
        pltpu.make_async_copy(
            txt_blk.at[0], out_hbm.at[pl.ds(0, L), pl.ds(D, L)],
            sem_out).wait()

    for j in range(BPW // TCHUNK):
        pltpu.make_async_copy(
            title_tab_hbm.at[pl.ds(0, TCHUNK)],
            title_rows_v.at[pl.ds(j * TCHUNK, TCHUNK)], sem_title).wait()
    pltpu.sync_copy(title_rows_v, out_hbm.at[pl.ds(tbase, BPW), pl.ds(0, D)])


_sc_call = pl.kernel(
    _body,
    out_type=jax.ShapeDtypeStruct((B, 2 * D), jnp.float32),
    mesh=plsc.VectorSubcoreMesh(
        core_axis_name="c", subcore_axis_name="s",
        num_cores=NC, num_subcores=NS),
    scratch_types=[
        pltpu.VMEM((V2, NP), jnp.int32),
        pltpu.VMEM((SEQ, GR), jnp.int32),
        pltpu.VMEM((GR,), jnp.float32),
        pltpu.VMEM((BPW,), jnp.int32),
        pltpu.VMEM((BPW, D), jnp.float32),
        pltpu.VMEM((2, L, L), jnp.float32),
        pltpu.SemaphoreType.DMA,
        pltpu.SemaphoreType.DMA,
    ],
    compiler_params=pltpu.CompilerParams(
        use_tc_tiling_on_sc=False, needs_layout_passes=False),
)


@jax.jit
def kernel(title_idx, text_tokens, title_table, text_table):
    ti = title_idx.astype(jnp.int32)
    tokT = text_tokens.astype(jnp.int32).T
    aug = jnp.concatenate(
        [text_table, jnp.zeros((V2 - TEXT_V, D), jnp.float32)], axis=0)
    pairs = lax.bitcast_convert_type(
        aug.astype(jnp.bfloat16).reshape(V2, D // 2, 2), jnp.int32)
    return _sc_call(ti, tokT, title_table, pairs[:, :NP], pairs[:, NP:])
